# Initial kernel scaffold; baseline (speedup 1.0000x reference)
import jax
import jax.numpy as jnp
from jax.experimental import pallas as pl


def kernel(user_emb, item_emb, edge_index, edge_vals):
    raise NotImplementedError("write your pallas kernel here")


# speedup vs baseline: 2.8929x; 2.8929x over previous
import functools

import jax
import jax.numpy as jnp
from jax import lax
from jax.experimental import pallas as pl
from jax.experimental.pallas import tpu as pltpu
from jax.experimental.pallas import tpu_sc as plsc

N_USERS = 25000
N_ITEMS = 25000
N_NODES = N_USERS + N_ITEMS
EMB = 64
N_EDGES = 800000

NC = 2
NS = 16
LANES = 16

HALF = N_NODES // NC
TRASH = HALF
ACC_ROWS = 26624
ZSTRIPE = ACC_ROWS // NS

CHUNK = 128
BLK = 2048
CPB = BLK // CHUNK
EPT = 51200
NBLK = EPT // BLK
E_PAD = NS * EPT

FLUSH_ROWS = 128
FLUSH_CHUNKS = -(-HALF // FLUSH_ROWS)
FLUSH_TAIL = HALF - (FLUSH_CHUNKS - 1) * FLUSH_ROWS

_MESH = plsc.VectorSubcoreMesh(core_axis_name="c", subcore_axis_name="s")


def _propagate_body(ego, src, dst, val, out,
                    src_b, dst_b, val_b, dst_v, rows_v, acc, sem):
    c = lax.axis_index("c")
    s = lax.axis_index("s")
    base_node = c * HALF

    @pl.loop(0, CHUNK)
    def _zero_rows(r):
        for q in range(EMB // LANES):
            rows_v[r, pl.ds(q * LANES, LANES)] = jnp.zeros((LANES,), jnp.float32)

    @pl.loop(0, ZSTRIPE // CHUNK)
    def _zero_acc(k):
        pltpu.sync_copy(rows_v, acc.at[pl.ds(s * ZSTRIPE + k * CHUNK, CHUNK)])

    plsc.subcore_barrier()

    @pl.loop(0, NBLK)
    def _block(b):
        ebase = s * EPT + b * BLK
        pltpu.sync_copy(src.at[pl.ds(ebase, BLK)], src_b)
        pltpu.sync_copy(dst.at[pl.ds(ebase, BLK)], dst_b)
        pltpu.sync_copy(val.at[pl.ds(ebase, BLK)], val_b)

        @pl.loop(0, CPB)
        def _chunk(k):
            koff = k * CHUNK
            pltpu.async_copy(ego.at[src_b.at[pl.ds(koff, CHUNK)]], rows_v,
                             sem).wait()
            for i in range(CHUNK // LANES):
                d16 = dst_b[pl.ds(koff + i * LANES, LANES)]
                inr = (d16 >= base_node) & (d16 < base_node + HALF)
                dst_v[pl.ds(i * LANES, LANES)] = jnp.where(
                    inr, d16 - base_node, TRASH)
            for i in range(CHUNK // LANES):
                v16 = val_b[pl.ds(koff + i * LANES, LANES)]
                for j in range(LANES):
                    e = i * LANES + j
                    vv = jnp.full((LANES,), v16[j], jnp.float32)
                    for q in range(EMB // LANES):
                        rows_v[e, pl.ds(q * LANES, LANES)] = (
                            rows_v[e, pl.ds(q * LANES, LANES)] * vv)
            pltpu.sync_copy(rows_v, acc.at[dst_v], add=True)

    plsc.subcore_barrier()

    nk = jnp.where(s < (FLUSH_CHUNKS % NS), FLUSH_CHUNKS // NS + 1,
                   FLUSH_CHUNKS // NS)

    @pl.loop(0, nk)
    def _flush(k):
        cid = s + k * NS
        off = cid * FLUSH_ROWS

        @pl.when(cid < FLUSH_CHUNKS - 1)
        def _full():
            pltpu.sync_copy(acc.at[pl.ds(off, FLUSH_ROWS)], rows_v)
            pltpu.sync_copy(rows_v, out.at[pl.ds(base_node + off, FLUSH_ROWS)])

        @pl.when(cid == FLUSH_CHUNKS - 1)
        def _tail():
            pltpu.sync_copy(acc.at[pl.ds(off, FLUSH_TAIL)],
                            rows_v.at[pl.ds(0, FLUSH_TAIL)])
            pltpu.sync_copy(rows_v.at[pl.ds(0, FLUSH_TAIL)],
                            out.at[pl.ds(base_node + off, FLUSH_TAIL)])


_propagate = functools.partial(
    pl.kernel,
    out_type=jax.ShapeDtypeStruct((N_NODES, EMB), jnp.float32),
    mesh=_MESH,
    scratch_types=[
        pltpu.VMEM((BLK,), jnp.int32),
        pltpu.VMEM((BLK,), jnp.int32),
        pltpu.VMEM((BLK,), jnp.float32),
        pltpu.VMEM((CHUNK,), jnp.int32),
        pltpu.VMEM((CHUNK, EMB), jnp.float32),
        pltpu.VMEM_SHARED((ACC_ROWS, EMB), jnp.float32),
        pltpu.SemaphoreType.DMA,
    ],
    compiler_params=pltpu.CompilerParams(use_tc_tiling_on_sc=False),
)(_propagate_body)


def _mean3_body(a, b, c, o):
    o[...] = (a[...] + b[...] + c[...]) * jnp.float32(1.0 / 3.0)


def _mean3(e1, e2, e3):
    n = N_NODES * EMB // 128
    blk = n // 25
    spec = pl.BlockSpec((blk, 128), lambda i: (i, 0))
    r = pl.pallas_call(
        _mean3_body,
        out_shape=jax.ShapeDtypeStruct((n, 128), jnp.float32),
        grid=(25,),
        in_specs=[spec, spec, spec],
        out_specs=spec,
    )(e1.reshape(n, 128), e2.reshape(n, 128), e3.reshape(n, 128))
    return r.reshape(N_NODES, EMB)


def kernel(user_emb, item_emb, edge_index, edge_vals):
    ego0 = jnp.concatenate([user_emb, item_emb], axis=0)
    pad = E_PAD - N_EDGES
    src = jnp.concatenate([edge_index[0], jnp.zeros((pad,), jnp.int32)])
    dst = jnp.concatenate([edge_index[1], jnp.zeros((pad,), jnp.int32)])
    val = jnp.concatenate([edge_vals, jnp.zeros((pad,), jnp.float32)])

    e1 = _propagate(ego0, src, dst, val)
    e2 = _propagate(e1, src, dst, val)
    e3 = _propagate(e2, src, dst, val)

    mean = _mean3(e1, e2, e3)
    return (mean[:N_USERS], mean[N_USERS:])


# speedup vs baseline: 4.5303x; 1.5660x over previous
import functools

import jax
import jax.numpy as jnp
from jax import lax
from jax.experimental import pallas as pl
from jax.experimental.pallas import tpu as pltpu
from jax.experimental.pallas import tpu_sc as plsc

N_USERS = 25000
N_ITEMS = 25000
N_NODES = N_USERS + N_ITEMS
EMB = 64
N_EDGES = 800000

NC = 2
NS = 16
LANES = 16

COLS = EMB // NC
QR = COLS // LANES

CHUNK = 128
BLK = 2048
CPB = BLK // CHUNK
EPT = 51200
NBLK = EPT // BLK
E_PAD = NS * EPT

ZSTRIPE = 3200
ACC_ROWS = NS * ZSTRIPE

FLUSH_ROWS = 128
FLUSH_CHUNKS = -(-N_NODES // FLUSH_ROWS)
FLUSH_TAIL = N_NODES - (FLUSH_CHUNKS - 1) * FLUSH_ROWS

_MESH = plsc.VectorSubcoreMesh(core_axis_name="c", subcore_axis_name="s")


def _propagate_body(ego, src, dst, val, out,
                    src_b, dst_b, val_b, dst_v, rows_v, acc, sem):
    c = lax.axis_index("c")
    s = lax.axis_index("s")
    row_base = c * N_NODES

    @pl.loop(0, CHUNK)
    def _zero_rows(r):
        for q in range(QR):
            rows_v[r, pl.ds(q * LANES, LANES)] = jnp.zeros((LANES,), jnp.float32)

    @pl.loop(0, ZSTRIPE // CHUNK)
    def _zero_acc(k):
        pltpu.sync_copy(rows_v, acc.at[pl.ds(s * ZSTRIPE + k * CHUNK, CHUNK)])

    plsc.subcore_barrier()

    @pl.loop(0, NBLK)
    def _block(b):
        ebase = s * EPT + b * BLK
        pltpu.sync_copy(src.at[pl.ds(ebase, BLK)], src_b)
        pltpu.sync_copy(dst.at[pl.ds(ebase, BLK)], dst_b)
        pltpu.sync_copy(val.at[pl.ds(ebase, BLK)], val_b)
        bias = jnp.full((LANES,), row_base, jnp.int32)

        @pl.loop(0, BLK // LANES)
        def _bias(i):
            src_b[pl.ds(i * LANES, LANES)] = (
                src_b[pl.ds(i * LANES, LANES)] + bias)

        @pl.loop(0, CPB)
        def _chunk(k):
            koff = k * CHUNK
            pltpu.async_copy(ego.at[src_b.at[pl.ds(koff, CHUNK)]],
                             rows_v, sem).wait()
            for i in range(CHUNK // LANES):
                dst_v[pl.ds(i * LANES, LANES)] = (
                    dst_b[pl.ds(koff + i * LANES, LANES)])
            for i in range(CHUNK // LANES):
                v16 = val_b[pl.ds(koff + i * LANES, LANES)]
                for j in range(LANES):
                    e = i * LANES + j
                    vv = jnp.full((LANES,), v16[j], jnp.float32)
                    for q in range(QR):
                        rows_v[e, pl.ds(q * LANES, LANES)] = (
                            rows_v[e, pl.ds(q * LANES, LANES)] * vv)
            pltpu.sync_copy(rows_v, acc.at[dst_v], add=True)

    plsc.subcore_barrier()

    nk = jnp.where(s < (FLUSH_CHUNKS % NS), FLUSH_CHUNKS // NS + 1,
                   FLUSH_CHUNKS // NS)

    @pl.loop(0, nk)
    def _flush(k):
        cid = s + k * NS
        off = cid * FLUSH_ROWS

        @pl.when(cid < FLUSH_CHUNKS - 1)
        def _full():
            pltpu.sync_copy(acc.at[pl.ds(off, FLUSH_ROWS)], rows_v)
            pltpu.sync_copy(rows_v, out.at[pl.ds(row_base + off, FLUSH_ROWS)])

        @pl.when(cid == FLUSH_CHUNKS - 1)
        def _tail():
            pltpu.sync_copy(acc.at[pl.ds(off, FLUSH_TAIL)],
                            rows_v.at[pl.ds(0, FLUSH_TAIL)])
            pltpu.sync_copy(rows_v.at[pl.ds(0, FLUSH_TAIL)],
                            out.at[pl.ds(row_base + off, FLUSH_TAIL)])


_propagate = functools.partial(
    pl.kernel,
    out_type=jax.ShapeDtypeStruct((NC * N_NODES, COLS), jnp.float32),
    mesh=_MESH,
    scratch_types=[
        pltpu.VMEM((BLK,), jnp.int32),
        pltpu.VMEM((BLK,), jnp.int32),
        pltpu.VMEM((BLK,), jnp.float32),
        pltpu.VMEM((CHUNK,), jnp.int32),
        pltpu.VMEM((CHUNK, COLS), jnp.float32),
        pltpu.VMEM_SHARED((ACC_ROWS, COLS), jnp.float32),
        pltpu.SemaphoreType.DMA,
    ],
    compiler_params=pltpu.CompilerParams(use_tc_tiling_on_sc=False),
)(_propagate_body)


def _mean3_body(a, b, c, o):
    o[...] = (a[...] + b[...] + c[...]) * jnp.float32(1.0 / 3.0)


def _mean3(e1, e2, e3):
    n = NC * N_NODES * COLS // 128
    r = pl.pallas_call(
        _mean3_body,
        out_shape=jax.ShapeDtypeStruct((n, 128), jnp.float32),
    )(e1.reshape(n, 128), e2.reshape(n, 128), e3.reshape(n, 128))
    return r.reshape(NC * N_NODES, COLS)


def kernel(user_emb, item_emb, edge_index, edge_vals):
    ego0 = jnp.concatenate([user_emb, item_emb], axis=0)
    ego_cat = jnp.concatenate([ego0[:, :COLS], ego0[:, COLS:]], axis=0)
    pad = E_PAD - N_EDGES
    src = jnp.concatenate([edge_index[0], jnp.zeros((pad,), jnp.int32)])
    dst = jnp.concatenate([edge_index[1], jnp.zeros((pad,), jnp.int32)])
    val = jnp.concatenate([edge_vals, jnp.zeros((pad,), jnp.float32)])

    e1 = _propagate(ego_cat, src, dst, val)
    e2 = _propagate(e1, src, dst, val)
    e3 = _propagate(e2, src, dst, val)

    mean = _mean3(e1, e2, e3)
    mean = jnp.concatenate([mean[:N_NODES], mean[N_NODES:]], axis=1)
    return (mean[:N_USERS], mean[N_USERS:])


# speedup vs baseline: 6.9801x; 1.5408x over previous
import functools

import jax
import jax.numpy as jnp
from jax import lax
from jax.experimental import pallas as pl
from jax.experimental.pallas import tpu as pltpu
from jax.experimental.pallas import tpu_sc as plsc

N_USERS = 25000
N_ITEMS = 25000
N_NODES = N_USERS + N_ITEMS
EMB = 64
N_EDGES = 800000

NC = 2
NS = 16
LANES = 16

COLS = EMB // NC
QR = COLS // LANES

CHUNK = 128
BLK = 2048
CPB = BLK // CHUNK
EPT = 51200
NBLK = EPT // BLK
E_PAD = NS * EPT

ZSTRIPE = 3200
ACC_ROWS = NS * ZSTRIPE

FLUSH_ROWS = 128
FLUSH_CHUNKS = -(-N_NODES // FLUSH_ROWS)
FLUSH_TAIL = N_NODES - (FLUSH_CHUNKS - 1) * FLUSH_ROWS

_MESH = plsc.VectorSubcoreMesh(core_axis_name="c", subcore_axis_name="s")


def _propagate_body(ego, src, dst, val, out,
                    src_b, dst_b, val_b,
                    dv0, dv1, dv2, dv3, rv0, rv1, rv2, rv3,
                    acc,
                    gs0, gs1, gs2, gs3, ss0, ss1, ss2, ss3):
    c = lax.axis_index("c")
    s = lax.axis_index("s")
    row_base = c * N_NODES
    dstv = [dv0, dv1, dv2, dv3]
    rows = [rv0, rv1, rv2, rv3]
    gsem = [gs0, gs1, gs2, gs3]
    ssem = [ss0, ss1, ss2, ss3]
    rows_v = rv0

    def _gather(cur, bb):
        return pltpu.make_async_copy(
            ego.at[src_b.at[pl.ds(cur * CHUNK, CHUNK)]], rows[bb], gsem[bb])

    def _scatter_start(bb):
        pltpu.async_copy(rows[bb], acc.at[dstv[bb]], ssem[bb], add=True)

    def _scatter_wait(bb):
        pltpu.make_async_copy(rows[bb], acc.at[dstv[bb]], ssem[bb]).wait()

    @pl.loop(0, CHUNK)
    def _zero_rows(r):
        for q in range(QR):
            rows_v[r, pl.ds(q * LANES, LANES)] = jnp.zeros((LANES,), jnp.float32)

    @pl.loop(0, ZSTRIPE // CHUNK)
    def _zero_acc(k):
        pltpu.sync_copy(rows_v, acc.at[pl.ds(s * ZSTRIPE + k * CHUNK, CHUNK)])

    plsc.subcore_barrier()

    @pl.loop(0, NBLK)
    def _block(b):
        ebase = s * EPT + b * BLK
        pltpu.sync_copy(src.at[pl.ds(ebase, BLK)], src_b)
        pltpu.sync_copy(dst.at[pl.ds(ebase, BLK)], dst_b)
        pltpu.sync_copy(val.at[pl.ds(ebase, BLK)], val_b)
        bias = jnp.full((LANES,), row_base, jnp.int32)

        @pl.loop(0, BLK // LANES)
        def _bias(i):
            src_b[pl.ds(i * LANES, LANES)] = (
                src_b[pl.ds(i * LANES, LANES)] + bias)

        for bb in range(3):
            _gather(bb, bb).start()

        @pl.loop(0, CPB // 4)
        def _chunk4(k4):
            for bb in range(4):
                cur = k4 * 4 + bb
                koff = cur * CHUNK
                _gather(cur, bb).wait()
                for i in range(CHUNK // LANES):
                    dstv[bb][pl.ds(i * LANES, LANES)] = (
                        dst_b[pl.ds(koff + i * LANES, LANES)])
                for i in range(CHUNK // LANES):
                    v16 = val_b[pl.ds(koff + i * LANES, LANES)]
                    for j in range(LANES):
                        e = i * LANES + j
                        vv = jnp.full((LANES,), v16[j], jnp.float32)
                        for q in range(QR):
                            rows[bb][e, pl.ds(q * LANES, LANES)] = (
                                rows[bb][e, pl.ds(q * LANES, LANES)] * vv)
                _scatter_start(bb)
                nb = (bb + 3) % 4

                @pl.when(cur + 3 < CPB)
                def _refill():
                    @pl.when(cur >= 1)
                    def _drain():
                        _scatter_wait(nb)

                    _gather(cur + 3, nb).start()

        for bb in range(4):
            _scatter_wait(bb)

    plsc.subcore_barrier()

    nk = jnp.where(s < (FLUSH_CHUNKS % NS), FLUSH_CHUNKS // NS + 1,
                   FLUSH_CHUNKS // NS)

    @pl.loop(0, nk)
    def _flush(k):
        cid = s + k * NS
        off = cid * FLUSH_ROWS

        @pl.when(cid < FLUSH_CHUNKS - 1)
        def _full():
            pltpu.sync_copy(acc.at[pl.ds(off, FLUSH_ROWS)], rows_v)
            pltpu.sync_copy(rows_v, out.at[pl.ds(row_base + off, FLUSH_ROWS)])

        @pl.when(cid == FLUSH_CHUNKS - 1)
        def _tail():
            pltpu.sync_copy(acc.at[pl.ds(off, FLUSH_TAIL)],
                            rows_v.at[pl.ds(0, FLUSH_TAIL)])
            pltpu.sync_copy(rows_v.at[pl.ds(0, FLUSH_TAIL)],
                            out.at[pl.ds(row_base + off, FLUSH_TAIL)])


_propagate = functools.partial(
    pl.kernel,
    out_type=jax.ShapeDtypeStruct((NC * N_NODES, COLS), jnp.float32),
    mesh=_MESH,
    scratch_types=(
        [
            pltpu.VMEM((BLK,), jnp.int32),
            pltpu.VMEM((BLK,), jnp.int32),
            pltpu.VMEM((BLK,), jnp.float32),
        ]
        + [pltpu.VMEM((CHUNK,), jnp.int32)] * 4
        + [pltpu.VMEM((CHUNK, COLS), jnp.float32)] * 4
        + [pltpu.VMEM_SHARED((ACC_ROWS, COLS), jnp.float32)]
        + [pltpu.SemaphoreType.DMA] * 8
    ),
    compiler_params=pltpu.CompilerParams(use_tc_tiling_on_sc=False),
)(_propagate_body)


def _mean3_body(a, b, c, o):
    o[...] = (a[...] + b[...] + c[...]) * jnp.float32(1.0 / 3.0)


def _mean3(e1, e2, e3):
    n = NC * N_NODES * COLS // 128
    r = pl.pallas_call(
        _mean3_body,
        out_shape=jax.ShapeDtypeStruct((n, 128), jnp.float32),
    )(e1.reshape(n, 128), e2.reshape(n, 128), e3.reshape(n, 128))
    return r.reshape(NC * N_NODES, COLS)


def kernel(user_emb, item_emb, edge_index, edge_vals):
    ego0 = jnp.concatenate([user_emb, item_emb], axis=0)
    ego_cat = jnp.concatenate([ego0[:, :COLS], ego0[:, COLS:]], axis=0)
    pad = E_PAD - N_EDGES
    src = jnp.concatenate([edge_index[0], jnp.zeros((pad,), jnp.int32)])
    dst = jnp.concatenate([edge_index[1], jnp.zeros((pad,), jnp.int32)])
    val = jnp.concatenate([edge_vals, jnp.zeros((pad,), jnp.float32)])

    e1 = _propagate(ego_cat, src, dst, val)
    e2 = _propagate(e1, src, dst, val)
    e3 = _propagate(e2, src, dst, val)

    mean = _mean3(e1, e2, e3)
    mean = jnp.concatenate([mean[:N_NODES], mean[N_NODES:]], axis=1)
    return (mean[:N_USERS], mean[N_USERS:])


# speedup vs baseline: 7.2206x; 1.0345x over previous
import functools

import jax
import jax.numpy as jnp
from jax import lax
from jax.experimental import pallas as pl
from jax.experimental.pallas import tpu as pltpu
from jax.experimental.pallas import tpu_sc as plsc

N_USERS = 25000
N_ITEMS = 25000
N_NODES = N_USERS + N_ITEMS
EMB = 64
N_EDGES = 800000

NC = 2
NS = 16
LANES = 16

COLS = EMB // NC
QR = COLS // LANES

CHUNK = 128
BLK = 2048
CPB = BLK // CHUNK
EPT = 51200
NBLK = EPT // BLK
E_PAD = NS * EPT

ZCH = 64
ZSTRIPE = 3136
ACC_ROWS = NS * ZSTRIPE

FLUSH_ROWS = 128
FLUSH_CHUNKS = -(-N_NODES // FLUSH_ROWS)
FLUSH_TAIL = N_NODES - (FLUSH_CHUNKS - 1) * FLUSH_ROWS

_MESH = plsc.VectorSubcoreMesh(core_axis_name="c", subcore_axis_name="s")


def _propagate_body(ego, src, dst, val, out,
                    sb0, db0, vb0, sb1, db1, vb1,
                    dv0, dv1, dv2, dv3, rv0, rv1, rv2, rv3,
                    acc,
                    gs0, gs1, gs2, gs3, ss0, ss1, ss2, ss3, st0, st1):
    c = lax.axis_index("c")
    s = lax.axis_index("s")
    row_base = c * N_NODES
    stage = [(sb0, db0, vb0), (sb1, db1, vb1)]
    stsem = [st0, st1]
    dstv = [dv0, dv1, dv2, dv3]
    rows = [rv0, rv1, rv2, rv3]
    gsem = [gs0, gs1, gs2, gs3]
    ssem = [ss0, ss1, ss2, ss3]
    rows_v = rv0

    def _stage_start(b, p):
        sb, db, vb = stage[p]
        ebase = s * EPT + b * BLK
        pltpu.async_copy(src.at[pl.ds(ebase, BLK)], sb, stsem[p])
        pltpu.async_copy(dst.at[pl.ds(ebase, BLK)], db, stsem[p])
        pltpu.async_copy(val.at[pl.ds(ebase, BLK)], vb, stsem[p])

    def _stage_wait(p):
        sb, db, vb = stage[p]
        pltpu.make_async_copy(src.at[pl.ds(0, BLK)], sb, stsem[p]).wait()
        pltpu.make_async_copy(dst.at[pl.ds(0, BLK)], db, stsem[p]).wait()
        pltpu.make_async_copy(val.at[pl.ds(0, BLK)], vb, stsem[p]).wait()

    def _gather(sb, cur, bb):
        return pltpu.make_async_copy(
            ego.at[sb.at[pl.ds(cur * CHUNK, CHUNK)]], rows[bb], gsem[bb])

    def _scatter_start(bb):
        pltpu.async_copy(rows[bb], acc.at[dstv[bb]], ssem[bb], add=True)

    def _scatter_wait(bb):
        pltpu.make_async_copy(rows[bb], acc.at[dstv[bb]], ssem[bb]).wait()

    _stage_start(0, 0)

    @pl.loop(0, ZCH)
    def _zero_rows(r):
        for q in range(QR):
            rows_v[r, pl.ds(q * LANES, LANES)] = jnp.zeros((LANES,), jnp.float32)

    @pl.loop(0, ZSTRIPE // ZCH)
    def _zero_start(k):
        pltpu.async_copy(rows_v.at[pl.ds(0, ZCH)],
                         acc.at[pl.ds(s * ZSTRIPE + k * ZCH, ZCH)], gs0)

    @pl.loop(0, ZSTRIPE // ZCH)
    def _zero_drain(k):
        pltpu.make_async_copy(
            rows_v.at[pl.ds(0, ZCH)],
            acc.at[pl.ds(s * ZSTRIPE + k * ZCH, ZCH)], gs0).wait()

    plsc.subcore_barrier()

    def _emit_block(b, p):
        sb, db, vb = stage[p]
        _stage_wait(p)

        @pl.when(b + 1 < NBLK)
        def _prefetch():
            _stage_start(b + 1, p ^ 1)

        bias = jnp.full((LANES,), row_base, jnp.int32)

        @pl.loop(0, BLK // LANES)
        def _bias(i):
            sb[pl.ds(i * LANES, LANES)] = sb[pl.ds(i * LANES, LANES)] + bias

        for bb in range(3):
            _gather(sb, bb, bb).start()

        @pl.loop(0, CPB // 4)
        def _chunk4(k4):
            for bb in range(4):
                cur = k4 * 4 + bb
                koff = cur * CHUNK
                _gather(sb, cur, bb).wait()
                for i in range(CHUNK // LANES):
                    dstv[bb][pl.ds(i * LANES, LANES)] = (
                        db[pl.ds(koff + i * LANES, LANES)])
                for i in range(CHUNK // LANES):
                    v16 = vb[pl.ds(koff + i * LANES, LANES)]
                    for j in range(LANES):
                        e = i * LANES + j
                        vv = jnp.full((LANES,), v16[j], jnp.float32)
                        for q in range(QR):
                            rows[bb][e, pl.ds(q * LANES, LANES)] = (
                                rows[bb][e, pl.ds(q * LANES, LANES)] * vv)
                _scatter_start(bb)
                nb = (bb + 3) % 4

                @pl.when(cur + 3 < CPB)
                def _refill():
                    @pl.when(cur >= 1)
                    def _drain():
                        _scatter_wait(nb)

                    _gather(sb, cur + 3, nb).start()

        for bb in range(4):
            _scatter_wait(bb)

    _emit_block(0, 0)

    @pl.loop(1, NBLK, step=2)
    def _block2(b2):
        _emit_block(b2, 1)
        _emit_block(b2 + 1, 0)

    plsc.subcore_barrier()

    nk = jnp.where(s < (FLUSH_CHUNKS % NS), FLUSH_CHUNKS // NS + 1,
                   FLUSH_CHUNKS // NS)

    @pl.loop(0, nk)
    def _flush(k):
        cid = s + k * NS
        off = cid * FLUSH_ROWS

        @pl.when(cid < FLUSH_CHUNKS - 1)
        def _full():
            pltpu.sync_copy(acc.at[pl.ds(off, FLUSH_ROWS)], rows_v)
            pltpu.sync_copy(rows_v, out.at[pl.ds(row_base + off, FLUSH_ROWS)])

        @pl.when(cid == FLUSH_CHUNKS - 1)
        def _tail():
            pltpu.sync_copy(acc.at[pl.ds(off, FLUSH_TAIL)],
                            rows_v.at[pl.ds(0, FLUSH_TAIL)])
            pltpu.sync_copy(rows_v.at[pl.ds(0, FLUSH_TAIL)],
                            out.at[pl.ds(row_base + off, FLUSH_TAIL)])


_propagate = functools.partial(
    pl.kernel,
    out_type=jax.ShapeDtypeStruct((NC * N_NODES, COLS), jnp.float32),
    mesh=_MESH,
    scratch_types=(
        [
            pltpu.VMEM((BLK,), jnp.int32),
            pltpu.VMEM((BLK,), jnp.int32),
            pltpu.VMEM((BLK,), jnp.float32),
            pltpu.VMEM((BLK,), jnp.int32),
            pltpu.VMEM((BLK,), jnp.int32),
            pltpu.VMEM((BLK,), jnp.float32),
        ]
        + [pltpu.VMEM((CHUNK,), jnp.int32)] * 4
        + [pltpu.VMEM((CHUNK, COLS), jnp.float32)] * 4
        + [pltpu.VMEM_SHARED((ACC_ROWS, COLS), jnp.float32)]
        + [pltpu.SemaphoreType.DMA] * 10
    ),
    compiler_params=pltpu.CompilerParams(use_tc_tiling_on_sc=False),
)(_propagate_body)


def _mean3_body(a, b, c, o):
    o[...] = (a[...] + b[...] + c[...]) * jnp.float32(1.0 / 3.0)


def _mean3(e1, e2, e3):
    n = NC * N_NODES * COLS // 128
    r = pl.pallas_call(
        _mean3_body,
        out_shape=jax.ShapeDtypeStruct((n, 128), jnp.float32),
    )(e1.reshape(n, 128), e2.reshape(n, 128), e3.reshape(n, 128))
    return r.reshape(NC * N_NODES, COLS)


def kernel(user_emb, item_emb, edge_index, edge_vals):
    ego0 = jnp.concatenate([user_emb, item_emb], axis=0)
    ego_cat = jnp.concatenate([ego0[:, :COLS], ego0[:, COLS:]], axis=0)
    pad = E_PAD - N_EDGES
    src = jnp.concatenate([edge_index[0], jnp.zeros((pad,), jnp.int32)])
    dst = jnp.concatenate([edge_index[1], jnp.zeros((pad,), jnp.int32)])
    val = jnp.concatenate([edge_vals, jnp.zeros((pad,), jnp.float32)])

    e1 = _propagate(ego_cat, src, dst, val)
    e2 = _propagate(e1, src, dst, val)
    e3 = _propagate(e2, src, dst, val)

    mean = _mean3(e1, e2, e3)
    mean = jnp.concatenate([mean[:N_NODES], mean[N_NODES:]], axis=1)
    return (mean[:N_USERS], mean[N_USERS:])


# speedup vs baseline: 9.4458x; 1.3082x over previous
import functools

import jax
import jax.numpy as jnp
import numpy as np
from jax import lax
from jax.experimental import pallas as pl
from jax.experimental.pallas import tpu as pltpu
from jax.experimental.pallas import tpu_sc as plsc

N_USERS = 25000
N_ITEMS = 25000
N_NODES = N_USERS + N_ITEMS
EMB = 64
N_EDGES = 800000

NC = 2
NS = 16
LANES = 16

COLS = EMB // NC
PCOLS = COLS // 2

CHUNK = 128
BLK = 2048
CPB = BLK // CHUNK
EPT = 51200
NBLK = EPT // BLK
E_PAD = NS * EPT

ZCH = 64
ZSTRIPE = 3136
ACC_ROWS = NS * ZSTRIPE

FLUSH_ROWS = 128
FLUSH_CHUNKS = -(-N_NODES // FLUSH_ROWS)
FLUSH_TAIL = N_NODES - (FLUSH_CHUNKS - 1) * FLUSH_ROWS

_HI = np.uint32(0xFFFF0000)
_RND = np.uint32(0x8000)

_MESH = plsc.VectorSubcoreMesh(core_axis_name="c", subcore_axis_name="s")


def _propagate_body(ego, src, dst2, val, out,
                    sb0, db0, vb0, sb1, db1, vb1,
                    pk0, pk1, pk2, pk3, fr0, fr1,
                    acc,
                    gs0, gs1, gs2, gs3, ss0, ss1, st0, st1):
    c = lax.axis_index("c")
    s = lax.axis_index("s")
    row_base = c * N_NODES
    stage = [(sb0, db0, vb0), (sb1, db1, vb1)]
    stsem = [st0, st1]
    pk = [pk0, pk1, pk2, pk3]
    fr = [fr0, fr1]
    gsem = [gs0, gs1, gs2, gs3]
    ssem = [ss0, ss1]

    def _stage_start(b, p):
        sb, db, vb = stage[p]
        ebase = s * EPT + b * BLK
        crow = s * (EPT // CHUNK) + b * CPB
        pltpu.async_copy(src.at[pl.ds(ebase, BLK)], sb, stsem[p])
        pltpu.async_copy(dst2.at[pl.ds(crow, CPB)], db, stsem[p])
        pltpu.async_copy(val.at[pl.ds(ebase, BLK)], vb, stsem[p])

    def _stage_wait(p):
        sb, db, vb = stage[p]
        pltpu.make_async_copy(src.at[pl.ds(0, BLK)], sb, stsem[p]).wait()
        pltpu.make_async_copy(dst2.at[pl.ds(0, CPB)], db, stsem[p]).wait()
        pltpu.make_async_copy(val.at[pl.ds(0, BLK)], vb, stsem[p]).wait()

    def _gather(sb, cur, bb):
        return pltpu.make_async_copy(
            ego.at[sb.at[pl.ds(cur * CHUNK, CHUNK)]], pk[bb], gsem[bb])

    def _scatter_start(db, cur, p2):
        pltpu.async_copy(fr[p2], acc.at[db.at[cur]], ssem[p2], add=True)

    def _scatter_wait(db, cur, p2):
        pltpu.make_async_copy(fr[p2], acc.at[db.at[cur]], ssem[p2]).wait()

    _stage_start(0, 0)

    @pl.loop(0, ZCH)
    def _zero_rows(r):
        for q in range(COLS // LANES):
            fr0[r, pl.ds(q * LANES, LANES)] = jnp.zeros((LANES,), jnp.float32)

    @pl.loop(0, ZSTRIPE // ZCH)
    def _zero_start(k):
        pltpu.async_copy(fr0.at[pl.ds(0, ZCH)],
                         acc.at[pl.ds(s * ZSTRIPE + k * ZCH, ZCH)], gs0)

    @pl.loop(0, ZSTRIPE // ZCH)
    def _zero_drain(k):
        pltpu.make_async_copy(
            fr0.at[pl.ds(0, ZCH)],
            acc.at[pl.ds(s * ZSTRIPE + k * ZCH, ZCH)], gs0).wait()

    plsc.subcore_barrier()

    def _emit_block(b, p):
        sb, db, vb = stage[p]
        _stage_wait(p)

        @pl.when(b + 1 < NBLK)
        def _prefetch():
            _stage_start(b + 1, p ^ 1)

        bias = jnp.full((LANES,), row_base, jnp.int32)

        @pl.loop(0, BLK // LANES)
        def _bias(i):
            sb[pl.ds(i * LANES, LANES)] = sb[pl.ds(i * LANES, LANES)] + bias

        for bb in range(3):
            _gather(sb, bb, bb).start()

        @pl.loop(0, CPB // 4)
        def _chunk4(k4):
            for bb in range(4):
                cur = k4 * 4 + bb
                p2 = bb % 2
                koff = cur * CHUNK
                _gather(sb, cur, bb).wait()

                @pl.when(cur >= 2)
                def _drain():
                    _scatter_wait(db, cur - 2, p2)

                for i in range(CHUNK // LANES):
                    v16 = vb[pl.ds(koff + i * LANES, LANES)]
                    for j in range(LANES):
                        e = i * LANES + j
                        vv = jnp.full((LANES,), v16[j], jnp.float32)
                        xi = plsc.bitcast(pk[bb][e, pl.ds(0, LANES)],
                                          jnp.uint32)
                        lo = plsc.bitcast(xi << 16, jnp.float32)
                        hi = plsc.bitcast(xi & _HI, jnp.float32)
                        fr[p2][e, pl.ds(0, LANES)] = lo * vv
                        fr[p2][e, pl.ds(LANES, LANES)] = hi * vv
                _scatter_start(db, cur, p2)

                @pl.when(cur + 3 < CPB)
                def _refill():
                    _gather(sb, cur + 3, (bb + 3) % 4).start()

        _scatter_wait(db, CPB - 2, 0)
        _scatter_wait(db, CPB - 1, 1)

    _emit_block(0, 0)

    @pl.loop(1, NBLK, step=2)
    def _block2(b2):
        _emit_block(b2, 1)
        _emit_block(b2 + 1, 0)

    plsc.subcore_barrier()

    nk = jnp.where(s < (FLUSH_CHUNKS % NS), FLUSH_CHUNKS // NS + 1,
                   FLUSH_CHUNKS // NS)

    def _pack_rows():
        @pl.loop(0, FLUSH_ROWS)
        def _pack(r):
            ai = plsc.bitcast(fr0[r, pl.ds(0, LANES)], jnp.uint32)
            bi = plsc.bitcast(fr0[r, pl.ds(LANES, LANES)], jnp.uint32)
            pa = (ai + _RND) >> 16
            pb = ((bi + _RND) >> 16) << 16
            pk0[r, pl.ds(0, LANES)] = plsc.bitcast(pa | pb, jnp.float32)

    @pl.loop(0, nk)
    def _flush(k):
        cid = s + k * NS
        off = cid * FLUSH_ROWS

        @pl.when(cid < FLUSH_CHUNKS - 1)
        def _full():
            pltpu.sync_copy(acc.at[pl.ds(off, FLUSH_ROWS)], fr0)
            _pack_rows()
            pltpu.sync_copy(pk0, out.at[pl.ds(row_base + off, FLUSH_ROWS)])

        @pl.when(cid == FLUSH_CHUNKS - 1)
        def _tail():
            pltpu.sync_copy(acc.at[pl.ds(off, FLUSH_TAIL)],
                            fr0.at[pl.ds(0, FLUSH_TAIL)])
            _pack_rows()
            pltpu.sync_copy(pk0.at[pl.ds(0, FLUSH_TAIL)],
                            out.at[pl.ds(row_base + off, FLUSH_TAIL)])


_propagate = functools.partial(
    pl.kernel,
    out_type=jax.ShapeDtypeStruct((NC * N_NODES, PCOLS), jnp.float32),
    mesh=_MESH,
    scratch_types=(
        [
            pltpu.VMEM((BLK,), jnp.int32),
            pltpu.VMEM((CPB, CHUNK), jnp.int32),
            pltpu.VMEM((BLK,), jnp.float32),
            pltpu.VMEM((BLK,), jnp.int32),
            pltpu.VMEM((CPB, CHUNK), jnp.int32),
            pltpu.VMEM((BLK,), jnp.float32),
        ]
        + [pltpu.VMEM((CHUNK, PCOLS), jnp.float32)] * 4
        + [pltpu.VMEM((CHUNK, COLS), jnp.float32)] * 2
        + [pltpu.VMEM_SHARED((ACC_ROWS, COLS), jnp.float32)]
        + [pltpu.SemaphoreType.DMA] * 8
    ),
    compiler_params=pltpu.CompilerParams(use_tc_tiling_on_sc=False,
                                         needs_layout_passes=False),
)(_propagate_body)


def _mean3_body(a, b, c, lo, hi):
    third = jnp.float32(1.0 / 3.0)

    def unp(x):
        xi = lax.bitcast_convert_type(x[...], jnp.uint32)
        l = lax.bitcast_convert_type(xi << 16, jnp.float32)
        h = lax.bitcast_convert_type(xi & _HI, jnp.float32)
        return l, h

    la, ha = unp(a)
    lb, hb = unp(b)
    lc, hc = unp(c)
    lo[...] = (la + lb + lc) * third
    hi[...] = (ha + hb + hc) * third


def _mean3(e1, e2, e3):
    n = NC * N_NODES * PCOLS // 128
    sds = jax.ShapeDtypeStruct((n, 128), jnp.float32)
    lo, hi = pl.pallas_call(
        _mean3_body,
        out_shape=(sds, sds),
    )(e1.reshape(n, 128), e2.reshape(n, 128), e3.reshape(n, 128))
    return (lo.reshape(NC, N_NODES, PCOLS), hi.reshape(NC, N_NODES, PCOLS))


def _pack_half(x):
    xi = lax.bitcast_convert_type(x, jnp.uint32)
    r = (xi + _RND) >> 16
    pk = r[:, :PCOLS] | (r[:, PCOLS:] << 16)
    return lax.bitcast_convert_type(pk, jnp.float32)


def kernel(user_emb, item_emb, edge_index, edge_vals):
    ego0 = jnp.concatenate([user_emb, item_emb], axis=0)
    ego_pk = jnp.concatenate(
        [_pack_half(ego0[:, :COLS]), _pack_half(ego0[:, COLS:])], axis=0)
    pad = E_PAD - N_EDGES
    src = jnp.concatenate([edge_index[0], jnp.zeros((pad,), jnp.int32)])
    dst = jnp.concatenate([edge_index[1], jnp.zeros((pad,), jnp.int32)])
    val = jnp.concatenate([edge_vals, jnp.zeros((pad,), jnp.float32)])
    dst2 = dst.reshape(E_PAD // CHUNK, CHUNK)

    e1 = _propagate(ego_pk, src, dst2, val)
    e2 = _propagate(e1, src, dst2, val)
    e3 = _propagate(e2, src, dst2, val)

    lo, hi = _mean3(e1, e2, e3)
    half0 = jnp.concatenate([lo[0], hi[0]], axis=1)
    half1 = jnp.concatenate([lo[1], hi[1]], axis=1)
    mean = jnp.concatenate([half0, half1], axis=1)
    return (mean[:N_USERS], mean[N_USERS:])
